# baseline (speedup 1.0000x reference)
import functools

import jax
import jax.numpy as jnp
from jax import lax
from jax.experimental import pallas as pl
from jax.experimental.pallas import tpu as pltpu
from jax.experimental.pallas import tpu_sc as plsc

_LANES = 16


@functools.partial(jax.jit, static_argnames=("nc", "ns"))
def _emb_gather(idx_t, table2, *, nc, ns):
    nw = nc * ns
    s_len, b0 = idx_t.shape
    _, dp = table2.shape
    d = dp // 2
    k = b0 // nw

    mesh = plsc.VectorSubcoreMesh(core_axis_name="c", subcore_axis_name="s")

    @functools.partial(
        pl.kernel,
        out_type=jax.ShapeDtypeStruct((s_len, d, b0), jnp.float32),
        mesh=mesh,
        scratch_types=[
            pltpu.VMEM((s_len, k), jnp.int32),
            pltpu.VMEM((2, k), jnp.int32),
            pltpu.VMEM((2, k), jnp.int32),
            pltpu.VMEM((k, dp), jnp.float32),
            pltpu.VMEM((k, dp), jnp.float32),
            pltpu.VMEM((d, k), jnp.float32),
            pltpu.SemaphoreType.DMA,
            pltpu.SemaphoreType.DMA,
        ],
        compiler_params=pltpu.CompilerParams(
            use_tc_tiling_on_sc=True, needs_layout_passes=False
        ),
    )
    def emb_kernel(
        idx_hbm, table_hbm, out_hbm,
        idx_v, gidx, offs, rows0, rows1, stage, sem0, sem1,
    ):
        wid = lax.axis_index("s") * nc + lax.axis_index("c")
        base = wid * k
        pltpu.sync_copy(idx_hbm.at[:, pl.ds(base, k)], idx_v)

        halves = ((rows0, sem0), (rows1, sem1))

        def fire(si, h):
            rows, sem = halves[h]
            for l in range(k // _LANES):
                vvec = idx_v[si, pl.ds(l * _LANES, _LANES)]
                gidx[h, pl.ds(l * _LANES, _LANES)] = vvec >> 1
                offs[h, pl.ds(l * _LANES, _LANES)] = (vvec & 1) * d
            pltpu.async_copy(table_hbm.at[gidx.at[h]], rows, sem)

        def drain_store(si, h):
            rows, sem = halves[h]
            pltpu.make_async_copy(table_hbm.at[gidx.at[h]], rows, sem).wait()
            for l in range(k // _LANES):
                rvec = lax.iota(jnp.int32, _LANES) + l * _LANES
                ovec = offs[h, pl.ds(l * _LANES, _LANES)]

                @pl.loop(0, d, unroll=16)
                def _tr(dr):
                    val = plsc.load_gather(rows, [rvec, ovec + dr])
                    stage[dr, pl.ds(l * _LANES, _LANES)] = val
            pltpu.sync_copy(
                stage, out_hbm.at[si].at[:, pl.ds(base, k)]
            )

        fire(0, 0)
        fire(1, 1)

        @pl.loop(0, s_len - 2, step=2)
        def _grp(i):
            for h in range(2):
                si = i + h
                drain_store(si, h)
                fire(si + 2, h)

        for si in (s_len - 2, s_len - 1):
            drain_store(si, si % 2)

    return emb_kernel(idx_t, table2)


def kernel(inputs, weight):
    b0, s = inputs.shape
    v, d = weight.shape
    info = plsc.get_sparse_core_info()
    nc, ns = info.num_cores, info.num_subcores
    idx_t = inputs.T.astype(jnp.int32)
    table2 = weight.reshape(v // 2, 2 * d)
    out = _emb_gather(idx_t, table2, nc=nc, ns=ns)
    return jnp.transpose(out, (2, 0, 1))


# speedup vs baseline: 2.3360x; 2.3360x over previous
import functools

import jax
import jax.numpy as jnp
from jax import lax
from jax.experimental import pallas as pl
from jax.experimental.pallas import tpu as pltpu
from jax.experimental.pallas import tpu_sc as plsc

_K = 128
_LANES = 16


@functools.partial(jax.jit, static_argnames=("nc", "ns"))
def _emb_gather(idx, table, *, nc, ns):
    nw = nc * ns
    _, n_chunks, k = idx.shape
    _, d = table.shape
    b = nw * n_chunks * k
    b_per_w = n_chunks * k

    mesh = plsc.VectorSubcoreMesh(core_axis_name="c", subcore_axis_name="s")

    @functools.partial(
        pl.kernel,
        out_type=jax.ShapeDtypeStruct((b, d), jnp.float32),
        mesh=mesh,
        scratch_types=[
            pltpu.VMEM((n_chunks, k), jnp.int32),
            pltpu.VMEM((k, d), jnp.float32),
            pltpu.VMEM((k, d), jnp.float32),
            pltpu.SemaphoreType.DMA,
            pltpu.SemaphoreType.DMA,
        ],
        compiler_params=pltpu.CompilerParams(use_tc_tiling_on_sc=True),
    )
    def emb_kernel(idx_hbm, table_hbm, out_hbm, idx_v, rows0, rows1, sem0, sem1):
        wid = lax.axis_index("s") * nc + lax.axis_index("c")
        base = wid * b_per_w
        pltpu.sync_copy(idx_hbm.at[wid], idx_v)

        halves = ((rows0, sem0), (rows1, sem1))

        def fire(gi, h):
            rows, sem = halves[h]
            for j16 in range(k // _LANES):
                vvec = idx_v[gi, pl.ds(j16 * _LANES, _LANES)]
                for j in range(_LANES):
                    r = j16 * _LANES + j
                    pltpu.async_copy(
                        table_hbm.at[pl.ds(vvec[j], 1)],
                        rows.at[pl.ds(r, 1)],
                        sem,
                    )

        def drain_store(gi, h):
            rows, sem = halves[h]
            pltpu.make_async_copy(
                table_hbm.at[pl.ds(0, k)], rows, sem
            ).wait()
            pltpu.sync_copy(rows, out_hbm.at[pl.ds(base + gi * k, k)])

        fire(0, 0)
        fire(1, 1)

        @pl.loop(0, n_chunks - 2, step=2)
        def _grp(i):
            for h in range(2):
                gi = i + h
                drain_store(gi, h)
                fire(gi + 2, h)

        for gi in (n_chunks - 2, n_chunks - 1):
            drain_store(gi, gi % 2)

    return emb_kernel(idx, table)


def kernel(inputs, weight):
    b0, s = inputs.shape
    _, d = weight.shape
    b = b0 * s
    info = plsc.get_sparse_core_info()
    nc, ns = info.num_cores, info.num_subcores
    nw = nc * ns
    idx = inputs.reshape(nw, b // (nw * _K), _K).astype(jnp.int32)
    table = jax.lax.optimization_barrier(weight)
    out = _emb_gather(idx, table, nc=nc, ns=ns)
    return out.reshape(b0, s, d)
